# baseline (speedup 1.0000x reference)
import functools

import jax
import jax.numpy as jnp
from jax import lax
from jax.experimental import pallas as pl
from jax.experimental.pallas import tpu as pltpu
from jax.experimental.pallas import tpu_sc as plsc

_R, _C = 16384, 4096
_N = _R * _C
_NSC = 2
_NSUB = 16
_CHALF = _C // _NSC
_NSLAB = 8
_SLABC = _CHALF // _NSLAB
_TROWS = _R // _NSUB
_CHR = 64
_CHUNK = _CHR * _SLABC
_HALF = _CHUNK // 2
_NCHUNK = _TROWS // _CHR
_SHIFT = 12


def _sc_gather(x2d, idx2):
    mesh = plsc.VectorSubcoreMesh(core_axis_name="c", subcore_axis_name="s")

    @functools.partial(
        pl.kernel,
        mesh=mesh,
        out_type=(
            jax.ShapeDtypeStruct((_R, _C), jnp.float32),
            jax.ShapeDtypeStruct((_N,), jnp.float32),
        ),
        scratch_types=[
            pltpu.VMEM((_CHR, _SLABC), jnp.int32),
            pltpu.VMEM((_CHR, _SLABC), jnp.int32),
            pltpu.VMEM((_HALF,), jnp.int32),
            pltpu.VMEM((_HALF,), jnp.int32),
            pltpu.VMEM((_HALF,), jnp.int32),
            pltpu.VMEM((_HALF,), jnp.int32),
            pltpu.VMEM((_HALF,), jnp.float32),
            pltpu.VMEM((_HALF,), jnp.float32),
            pltpu.VMEM((_HALF,), jnp.float32),
            pltpu.VMEM((_HALF,), jnp.float32),
            pltpu.SemaphoreType.DMA,
            pltpu.SemaphoreType.DMA,
            pltpu.SemaphoreType.DMA,
            pltpu.SemaphoreType.DMA,
            pltpu.SemaphoreType.DMA,
            pltpu.SemaphoreType.DMA,
            pltpu.SemaphoreType.DMA,
        ],
    )
    def k(x_hbm, idx_hbm, out_hbm, xl_hbm, idx_a, idx_b,
          fidx_a1, fidx_a2, fidx_b1, fidx_b2,
          data_a1, data_a2, data_b1, data_b2,
          sem_in, sem_out, sem_rl, sem_ga1, sem_ga2, sem_gb1, sem_gb2):
        cid = lax.axis_index("c")
        sid = lax.axis_index("s")
        col0 = cid * _CHALF
        row0 = sid * _TROWS
        lane = lax.iota(jnp.int32, 16)

        def relayout_start(slab):
            sc0 = col0 + slab * _SLABC

            def rrow(r, carry):
                row = sid * (_R // _NSUB) + r
                pltpu.make_async_copy(
                    x_hbm.at[row, pl.ds(sc0, _SLABC)],
                    xl_hbm.at[pl.ds(row * _C + sc0, _SLABC)],
                    sem_rl).start()
                return carry

            lax.fori_loop(0, _R // _NSUB, rrow, 0)

        def relayout_drain(slab):
            sc0 = col0 + slab * _SLABC

            def rrow(r, carry):
                row = sid * (_R // _NSUB) + r
                pltpu.make_async_copy(
                    x_hbm.at[row, pl.ds(sc0, _SLABC)],
                    xl_hbm.at[pl.ds(row * _C + sc0, _SLABC)],
                    sem_rl).wait()
                return carry

            lax.fori_loop(0, _R // _NSUB, rrow, 0)

        def idx_start(slab, c, idx_v):
            pltpu.make_async_copy(
                idx_hbm.at[pl.ds(row0 + c * _CHR, _CHR),
                           pl.ds(col0 + slab * _SLABC, _SLABC)],
                idx_v, sem_in).start()

        def idx_wait(slab, c, idx_v):
            pltpu.make_async_copy(
                idx_hbm.at[pl.ds(row0 + c * _CHR, _CHR),
                           pl.ds(col0 + slab * _SLABC, _SLABC)],
                idx_v, sem_in).wait()

        def fidx_compute(slab, idx_v, fidx_1, fidx_2):
            sc0 = col0 + slab * _SLABC
            hr = _CHR // 2

            def frow(e, carry):
                r = e >> 4
                cbase = sc0 + ((e & 15) << 4)
                sl = pl.ds(pl.multiple_of((e & 15) << 4, 16), 16)
                fsl = pl.ds(pl.multiple_of(e * 16, 16), 16)
                fidx_1[fsl] = (idx_v[r, sl] << _SHIFT) | (cbase + lane)
                fidx_2[fsl] = (idx_v[hr + r, sl] << _SHIFT) | (cbase + lane)
                return carry

            lax.fori_loop(0, _HALF // 16, frow, 0, unroll=8)

        def gather_start(fidx_v, data_v, sem):
            pltpu.make_async_copy(xl_hbm.at[fidx_v], data_v, sem).start()

        def gather_wait(fidx_v, data_v, sem):
            pltpu.make_async_copy(xl_hbm.at[fidx_v], data_v, sem).wait()

        def out_rows(slab, c, data_1, data_2, fire):
            sc0 = col0 + slab * _SLABC
            hr = _CHR // 2

            def orow1(r, carry):
                cp = pltpu.make_async_copy(
                    data_1.at[pl.ds(r * _SLABC, _SLABC)],
                    out_hbm.at[row0 + c * _CHR + r, pl.ds(sc0, _SLABC)],
                    sem_out)
                cp.start() if fire else cp.wait()
                return carry

            def orow2(r, carry):
                cp = pltpu.make_async_copy(
                    data_2.at[pl.ds(r * _SLABC, _SLABC)],
                    out_hbm.at[row0 + c * _CHR + hr + r, pl.ds(sc0, _SLABC)],
                    sem_out)
                cp.start() if fire else cp.wait()
                return carry

            lax.fori_loop(0, hr, orow1, 0)
            lax.fori_loop(0, hr, orow2, 0)

        def out_start(slab, c, data_1, data_2):
            out_rows(slab, c, data_1, data_2, True)

        def out_wait(slab, c, data_1, data_2):
            out_rows(slab, c, data_1, data_2, False)

        bufs_a = (idx_a, fidx_a1, fidx_a2, data_a1, data_a2, sem_ga1, sem_ga2)
        bufs_b = (idx_b, fidx_b1, fidx_b2, data_b1, data_b2, sem_gb1, sem_gb2)

        def half(slab, c, cur, nxt):
            (idx_c, fidx_c1, fidx_c2, data_c1, data_c2, sem_c1, sem_c2) = cur
            (idx_n, fidx_n1, fidx_n2, data_n1, data_n2, sem_n1, sem_n2) = nxt

            @pl.when(c + 1 < _NCHUNK)
            def _stage_next():
                idx_wait(slab, c + 1, idx_n)
                fidx_compute(slab, idx_n, fidx_n1, fidx_n2)

            @pl.when(c > 0)
            def _drain_prev_out():
                out_wait(slab, c - 1, data_n1, data_n2)

            @pl.when(c + 1 < _NCHUNK)
            def _fire_next():
                gather_start(fidx_n1, data_n1, sem_n1)
                gather_start(fidx_n2, data_n2, sem_n2)

            gather_wait(fidx_c1, data_c1, sem_c1)
            gather_wait(fidx_c2, data_c2, sem_c2)
            out_start(slab, c, data_c1, data_c2)

            @pl.when(c + 2 < _NCHUNK)
            def _prefetch():
                idx_start(slab, c + 2, idx_c)

        relayout_start(0)
        idx_start(0, 0, idx_a)
        idx_start(0, 1, idx_b)
        idx_wait(0, 0, idx_a)
        fidx_compute(0, idx_a, fidx_a1, fidx_a2)
        relayout_drain(0)
        plsc.subcore_barrier()

        def slab_body(slab, carry):
            @pl.when(slab + 1 < _NSLAB)
            def _relayout_next():
                relayout_start(slab + 1)

            gather_start(fidx_a1, data_a1, sem_ga1)
            gather_start(fidx_a2, data_a2, sem_ga2)

            def pair_body(cp, carry2):
                half(slab, 2 * cp, bufs_a, bufs_b)
                half(slab, 2 * cp + 1, bufs_b, bufs_a)
                return carry2

            lax.fori_loop(0, _NCHUNK // 2, pair_body, 0)
            out_wait(slab, _NCHUNK - 1, data_b1, data_b2)

            @pl.when(slab + 1 < _NSLAB)
            def _prep_next():
                idx_start(slab + 1, 0, idx_a)
                idx_start(slab + 1, 1, idx_b)
                idx_wait(slab + 1, 0, idx_a)
                fidx_compute(slab + 1, idx_a, fidx_a1, fidx_a2)
                relayout_drain(slab + 1)

            plsc.subcore_barrier()
            return carry

        lax.fori_loop(0, _NSLAB, slab_body, 0)

    return k(x2d, idx2)


def kernel(x, index):
    out, _ = _sc_gather(x, index)
    return out


# speedup vs baseline: 3.7509x; 3.7509x over previous
import functools

import jax
import jax.numpy as jnp
from jax import lax
from jax.experimental import pallas as pl
from jax.experimental.pallas import tpu as pltpu
from jax.experimental.pallas import tpu_sc as plsc

_R, _C = 16384, 4096
_N = _R * _C
_NW = 32
_WROWS = _R // _NW
_CR = 4
_CHUNK = _CR * _C
_HALF = _CHUNK // 2
_NCHUNK = _WROWS // _CR
_SHIFT = 12


def _sc_gather(x1d, idx2):
    mesh = plsc.VectorSubcoreMesh(core_axis_name="c", subcore_axis_name="s")

    @functools.partial(
        pl.kernel,
        mesh=mesh,
        out_type=jax.ShapeDtypeStruct((_R, _C), jnp.float32),
        scratch_types=[
            pltpu.VMEM((_CHUNK,), jnp.int32),
            pltpu.VMEM((_CHUNK,), jnp.int32),
            pltpu.VMEM((_HALF,), jnp.int32),
            pltpu.VMEM((_HALF,), jnp.int32),
            pltpu.VMEM((_HALF,), jnp.int32),
            pltpu.VMEM((_HALF,), jnp.int32),
            pltpu.VMEM((_HALF,), jnp.float32),
            pltpu.VMEM((_HALF,), jnp.float32),
            pltpu.VMEM((_HALF,), jnp.float32),
            pltpu.VMEM((_HALF,), jnp.float32),
            pltpu.SemaphoreType.DMA,
            pltpu.SemaphoreType.DMA,
            pltpu.SemaphoreType.DMA,
            pltpu.SemaphoreType.DMA,
            pltpu.SemaphoreType.DMA,
            pltpu.SemaphoreType.DMA,
        ],
    )
    def k(x_hbm, idx_hbm, out_hbm, idx_a, idx_b,
          fidx_a1, fidx_a2, fidx_b1, fidx_b2,
          data_a1, data_a2, data_b1, data_b2,
          sem_in, sem_out, sem_ga1, sem_ga2, sem_gb1, sem_gb2):
        wid = lax.axis_index("s") * 2 + lax.axis_index("c")
        base = wid * _WROWS
        lane = lax.iota(jnp.int32, 16)

        def idx_start(c, idx_v):
            for r in range(_CR):
                pltpu.make_async_copy(
                    idx_hbm.at[base + c * _CR + r],
                    idx_v.at[pl.ds(r * _C, _C)], sem_in).start()

        def idx_wait(c, idx_v):
            for r in range(_CR):
                pltpu.make_async_copy(
                    idx_hbm.at[base + c * _CR + r],
                    idx_v.at[pl.ds(r * _C, _C)], sem_in).wait()

        def fidx_compute(idx_v, fidx_1, fidx_2):
            def frow(r, carry):
                col = (lax.rem(r, _C // 16) << 4) + lane
                fidx_1[pl.ds(r * 16, 16)] = (
                    (idx_v[pl.ds(r * 16, 16)] << _SHIFT) | col)
                fidx_2[pl.ds(r * 16, 16)] = (
                    (idx_v[pl.ds(_HALF + r * 16, 16)] << _SHIFT) | col)
                return carry
            lax.fori_loop(0, _HALF // 16, frow, 0, unroll=8)

        def gather_start(fidx_v, data_v, sem):
            pltpu.make_async_copy(x_hbm.at[fidx_v], data_v, sem).start()

        def gather_wait(fidx_v, data_v, sem):
            pltpu.make_async_copy(x_hbm.at[fidx_v], data_v, sem).wait()

        def out_start(c, data_1, data_2):
            for r in range(_CR):
                d = data_1 if r < _CR // 2 else data_2
                o = (r % (_CR // 2)) * _C
                pltpu.make_async_copy(
                    d.at[pl.ds(o, _C)],
                    out_hbm.at[base + c * _CR + r], sem_out).start()

        def out_wait(c, data_1, data_2):
            for r in range(_CR):
                d = data_1 if r < _CR // 2 else data_2
                o = (r % (_CR // 2)) * _C
                pltpu.make_async_copy(
                    d.at[pl.ds(o, _C)],
                    out_hbm.at[base + c * _CR + r], sem_out).wait()

        idx_start(0, idx_a)
        idx_start(1, idx_b)
        idx_wait(0, idx_a)
        fidx_compute(idx_a, fidx_a1, fidx_a2)
        gather_start(fidx_a1, data_a1, sem_ga1)
        gather_start(fidx_a2, data_a2, sem_ga2)

        def half(c, cur, nxt):
            (idx_c, fidx_c1, fidx_c2, data_c1, data_c2, sem_c1, sem_c2) = cur
            (idx_n, fidx_n1, fidx_n2, data_n1, data_n2, sem_n1, sem_n2) = nxt

            @pl.when(c + 1 < _NCHUNK)
            def _stage_next():
                idx_wait(c + 1, idx_n)
                fidx_compute(idx_n, fidx_n1, fidx_n2)

            @pl.when(c > 0)
            def _drain_prev_out():
                out_wait(c - 1, data_n1, data_n2)

            @pl.when(c + 1 < _NCHUNK)
            def _fire_next():
                gather_start(fidx_n1, data_n1, sem_n1)
                gather_start(fidx_n2, data_n2, sem_n2)

            gather_wait(fidx_c1, data_c1, sem_c1)
            gather_wait(fidx_c2, data_c2, sem_c2)
            out_start(c, data_c1, data_c2)

            @pl.when(c + 2 < _NCHUNK)
            def _prefetch():
                idx_start(c + 2, idx_c)

        bufs_a = (idx_a, fidx_a1, fidx_a2, data_a1, data_a2, sem_ga1, sem_ga2)
        bufs_b = (idx_b, fidx_b1, fidx_b2, data_b1, data_b2, sem_gb1, sem_gb2)

        def pair_body(cp, carry):
            half(2 * cp, bufs_a, bufs_b)
            half(2 * cp + 1, bufs_b, bufs_a)
            return carry

        lax.fori_loop(0, _NCHUNK // 2, pair_body, 0)
        out_wait(_NCHUNK - 1, data_b1, data_b2)

    return k(x1d, idx2)


def kernel(x, index):
    x1d = x.reshape(_N)
    return _sc_gather(x1d, index)


# speedup vs baseline: 3.7546x; 1.0010x over previous
import functools

import jax
import jax.numpy as jnp
from jax import lax
from jax.experimental import pallas as pl
from jax.experimental.pallas import tpu as pltpu
from jax.experimental.pallas import tpu_sc as plsc
from jax.experimental.compute_on import compute_on

_R, _C = 16384, 4096
_N = _R * _C
_NW = 32
_WROWS = _R // _NW
_CR = 4
_CHUNK = _CR * _C
_HALF = _CHUNK // 2
_NCHUNK = _WROWS // _CR
_SHIFT = 12


def _sc_gather(x1d, idx2):
    mesh = plsc.VectorSubcoreMesh(core_axis_name="c", subcore_axis_name="s")

    @functools.partial(
        pl.kernel,
        mesh=mesh,
        out_type=jax.ShapeDtypeStruct((_R, _C), jnp.float32),
        scratch_types=[
            pltpu.VMEM((_CHUNK,), jnp.int32),
            pltpu.VMEM((_CHUNK,), jnp.int32),
            pltpu.VMEM((_HALF,), jnp.int32),
            pltpu.VMEM((_HALF,), jnp.int32),
            pltpu.VMEM((_HALF,), jnp.int32),
            pltpu.VMEM((_HALF,), jnp.int32),
            pltpu.VMEM((_HALF,), jnp.float32),
            pltpu.VMEM((_HALF,), jnp.float32),
            pltpu.VMEM((_HALF,), jnp.float32),
            pltpu.VMEM((_HALF,), jnp.float32),
            pltpu.SemaphoreType.DMA,
            pltpu.SemaphoreType.DMA,
            pltpu.SemaphoreType.DMA,
            pltpu.SemaphoreType.DMA,
            pltpu.SemaphoreType.DMA,
            pltpu.SemaphoreType.DMA,
        ],
    )
    def k(x_hbm, idx_hbm, out_hbm, idx_a, idx_b,
          fidx_a1, fidx_a2, fidx_b1, fidx_b2,
          data_a1, data_a2, data_b1, data_b2,
          sem_in, sem_out, sem_ga1, sem_ga2, sem_gb1, sem_gb2):
        wid = lax.axis_index("s") * 2 + lax.axis_index("c")
        base = wid * _WROWS
        lane = lax.iota(jnp.int32, 16)

        def idx_start(c, idx_v):
            for r in range(_CR):
                pltpu.make_async_copy(
                    idx_hbm.at[base + c * _CR + r],
                    idx_v.at[pl.ds(r * _C, _C)], sem_in).start()

        def idx_wait(c, idx_v):
            for r in range(_CR):
                pltpu.make_async_copy(
                    idx_hbm.at[base + c * _CR + r],
                    idx_v.at[pl.ds(r * _C, _C)], sem_in).wait()

        def fidx_compute(idx_v, fidx_1, fidx_2):
            def frow(r, carry):
                col = (lax.rem(r, _C // 16) << 4) + lane
                fidx_1[pl.ds(r * 16, 16)] = (
                    (idx_v[pl.ds(r * 16, 16)] << _SHIFT) | col)
                fidx_2[pl.ds(r * 16, 16)] = (
                    (idx_v[pl.ds(_HALF + r * 16, 16)] << _SHIFT) | col)
                return carry
            lax.fori_loop(0, _HALF // 16, frow, 0, unroll=8)

        def gather_start(fidx_v, data_v, sem):
            pltpu.make_async_copy(x_hbm.at[fidx_v], data_v, sem).start()

        def gather_wait(fidx_v, data_v, sem):
            pltpu.make_async_copy(x_hbm.at[fidx_v], data_v, sem).wait()

        def out_start(c, data_1, data_2):
            for r in range(_CR):
                d = data_1 if r < _CR // 2 else data_2
                o = (r % (_CR // 2)) * _C
                pltpu.make_async_copy(
                    d.at[pl.ds(o, _C)],
                    out_hbm.at[base + c * _CR + r], sem_out).start()

        def out_wait(c, data_1, data_2):
            for r in range(_CR):
                d = data_1 if r < _CR // 2 else data_2
                o = (r % (_CR // 2)) * _C
                pltpu.make_async_copy(
                    d.at[pl.ds(o, _C)],
                    out_hbm.at[base + c * _CR + r], sem_out).wait()

        idx_start(0, idx_a)
        idx_start(1, idx_b)
        idx_wait(0, idx_a)
        fidx_compute(idx_a, fidx_a1, fidx_a2)
        gather_start(fidx_a1, data_a1, sem_ga1)
        gather_start(fidx_a2, data_a2, sem_ga2)

        def half(c, cur, nxt):
            (idx_c, fidx_c1, fidx_c2, data_c1, data_c2, sem_c1, sem_c2) = cur
            (idx_n, fidx_n1, fidx_n2, data_n1, data_n2, sem_n1, sem_n2) = nxt

            @pl.when(c + 1 < _NCHUNK)
            def _stage_next():
                idx_wait(c + 1, idx_n)
                fidx_compute(idx_n, fidx_n1, fidx_n2)

            @pl.when(c > 0)
            def _drain_prev_out():
                out_wait(c - 1, data_n1, data_n2)

            @pl.when(c + 1 < _NCHUNK)
            def _fire_next():
                gather_start(fidx_n1, data_n1, sem_n1)
                gather_start(fidx_n2, data_n2, sem_n2)

            gather_wait(fidx_c1, data_c1, sem_c1)
            gather_wait(fidx_c2, data_c2, sem_c2)
            out_start(c, data_c1, data_c2)

            @pl.when(c + 2 < _NCHUNK)
            def _prefetch():
                idx_start(c + 2, idx_c)

        bufs_a = (idx_a, fidx_a1, fidx_a2, data_a1, data_a2, sem_ga1, sem_ga2)
        bufs_b = (idx_b, fidx_b1, fidx_b2, data_b1, data_b2, sem_gb1, sem_gb2)

        def pair_body(cp, carry):
            half(2 * cp, bufs_a, bufs_b)
            half(2 * cp + 1, bufs_b, bufs_a)
            return carry

        lax.fori_loop(0, _NCHUNK // 2, pair_body, 0)
        out_wait(_NCHUNK - 1, data_b1, data_b2)

    return k(x1d, idx2)


def kernel(x, index):
    with compute_on("device"):
        x1d = x.reshape(_N)
    return _sc_gather(x1d, index)
